# Initial kernel scaffold; baseline (speedup 1.0000x reference)
import functools

import jax
import jax.numpy as jnp
from jax import lax
from jax.experimental import pallas as pl
from jax.experimental.pallas import tpu as pltpu
from jax.experimental.pallas import tpu_sc as plsc

N = 10000
E = 320000
D_IN = 128
HID = 8
HEADS = 8
N_CLASSES = 2
N_GRAPHS = 64

NC = 2
NS = 16
NW = NC * NS
K = 128
NP = 10240
EP = NW * K * ((E + NW * K - 1) // (NW * K))
CHUNKS = EP // (NW * K)
RT = NP // NS

_f32 = jnp.float32
_i32 = jnp.int32


def _p1_body(x_ref, w1_ref, am_ref, bm_ref, h1_ref, ca_ref, cb_ref):
    h1 = jnp.dot(x_ref[...], w1_ref[...], preferred_element_type=_f32)
    h1_ref[...] = h1
    ca_ref[...] = jnp.dot(h1, am_ref[...], preferred_element_type=_f32)
    cb_ref[...] = jnp.dot(h1, bm_ref[...], preferred_element_type=_f32)


def _l1_body(src_hbm, dst_hbm, h1_hbm, ca_hbm, cb_hbm, z64_hbm, z16_hbm,
             acc_out, den_out,
             srcv, dstv, hrows, arows, brows, wbuf, acc_s, den_s,
             sem1, sem2, sem3):
    c = lax.axis_index("c")
    s = lax.axis_index("s")
    wid = s * NC + c
    rows0 = s * RT
    pltpu.sync_copy(z64_hbm.at[pl.ds(rows0, RT)], acc_s.at[pl.ds(rows0, RT)])
    pltpu.sync_copy(z16_hbm.at[pl.ds(rows0, RT)], den_s.at[pl.ds(rows0, RT)])
    plsc.subcore_barrier()

    lane = lax.broadcasted_iota(_i32, (16,), 0)
    qsel = lane >> 3
    base = wid * (EP // NW)

    @pl.loop(0, CHUNKS)
    def _chunk(i):
        off = base + i * K
        pltpu.sync_copy(src_hbm.at[pl.ds(off, K)], srcv)
        pltpu.sync_copy(dst_hbm.at[pl.ds(off, K)], dstv)
        ga = pltpu.async_copy(ca_hbm.at[srcv], arows, sem1)
        gb = pltpu.async_copy(cb_hbm.at[dstv], brows, sem2)
        gh = pltpu.async_copy(h1_hbm.at[srcv], hrows, sem3)
        ga.wait()
        gb.wait()
        gh.wait()

        @pl.loop(0, K)
        def _edge(e):
            t = arows[e] + brows[e]
            t = jnp.where(t >= 0, t, 0.2 * t)
            w = jnp.exp(t)
            wbuf[e] = w
            erep = jnp.full((16,), e, _i32)
            for q in range(4):
                wq = plsc.load_gather(wbuf, [erep, qsel + 2 * q])
                hrows[e, q] = hrows[e, q] * wq

        pltpu.sync_copy(wbuf, den_s.at[dstv], add=True)
        pltpu.sync_copy(hrows, acc_s.at[dstv], add=True)

    plsc.subcore_barrier()
    pltpu.sync_copy(acc_s.at[pl.ds(rows0, RT)], acc_out.at[c, pl.ds(rows0, RT)])
    pltpu.sync_copy(den_s.at[pl.ds(rows0, RT)], den_out.at[c, pl.ds(rows0, RT)])


def _p3_body(a0_ref, a1_ref, d0_ref, d1_ref, r16_ref, b1_ref, w2_ref,
             ma_ref, mb_ref, c3_ref, ta_ref, tb_ref):
    acc = a0_ref[...] + a1_ref[...]
    den = d0_ref[...] + d1_ref[...]
    deno = jnp.dot(den, r16_ref[...], preferred_element_type=_f32)
    h = acc / (deno + 1e-16) + b1_ref[...]
    h = jnp.where(h > 0, h, jnp.exp(h) - 1.0)
    h2 = jnp.dot(h, w2_ref[...], preferred_element_type=_f32)
    ta_ref[...] = jnp.dot(h2, ma_ref[...], preferred_element_type=_f32) + c3_ref[...]
    tb_ref[...] = jnp.dot(h2, mb_ref[...], preferred_element_type=_f32)


def _l2_body(src_hbm, dst_hbm, ta_hbm, tb_hbm, z16_hbm,
             acc_out,
             srcv, dstv, arows, brows, rowbuf, tmp, acc_s,
             sem1, sem2):
    c = lax.axis_index("c")
    s = lax.axis_index("s")
    wid = s * NC + c
    rows0 = s * RT
    pltpu.sync_copy(z16_hbm.at[pl.ds(rows0, RT)], acc_s.at[pl.ds(rows0, RT)])
    plsc.subcore_barrier()

    lane = lax.broadcasted_iota(_i32, (16,), 0)
    zero16 = jnp.zeros((16,), _i32)
    pat = jnp.where(lane == 0, 3, jnp.where(lane < 3, lane, 4))
    base = wid * (EP // NW)

    @pl.loop(0, CHUNKS)
    def _chunk(i):
        off = base + i * K
        pltpu.sync_copy(src_hbm.at[pl.ds(off, K)], srcv)
        pltpu.sync_copy(dst_hbm.at[pl.ds(off, K)], dstv)
        ga = pltpu.async_copy(ta_hbm.at[srcv], arows, sem1)
        gb = pltpu.async_copy(tb_hbm.at[dstv], brows, sem2)
        ga.wait()
        gb.wait()

        @pl.loop(0, K)
        def _edge(e):
            a = arows[e]
            t = a + brows[e]
            tmp[...] = t
            g0 = plsc.load_gather(tmp, [zero16])
            g0 = jnp.where(g0 >= 0, g0, 0.2 * g0)
            w = jnp.exp(g0)
            erep = jnp.full((16,), e, _i32)
            mult = plsc.load_gather(arows, [erep, pat])
            rowbuf[e] = w * mult

        pltpu.sync_copy(rowbuf, acc_s.at[dstv], add=True)

    plsc.subcore_barrier()
    pltpu.sync_copy(acc_s.at[pl.ds(rows0, RT)], acc_out.at[c, pl.ds(rows0, RT)])


def _p5_body(a0_ref, a1_ref, batch_ref, b2_ref, out_ref):
    acc = a0_ref[...] + a1_ref[...]
    den = acc[:, 0:1]
    o = acc[:, 1:3] / (den + 1e-16) + b2_ref[...]
    ids = batch_ref[...]
    g = lax.broadcasted_iota(_i32, (N_GRAPHS, 1), 0)
    mt = (g == ids).astype(_f32)
    sums = jnp.dot(mt, o, preferred_element_type=_f32)
    cnts = jnp.sum(mt, axis=1, keepdims=True)
    pooled = sums / jnp.maximum(cnts, 1.0)
    m = jnp.max(pooled, axis=1, keepdims=True)
    z = pooled - m
    out_ref[...] = z - jnp.log(jnp.sum(jnp.exp(z), axis=1, keepdims=True))


def _sds(shape, dtype=_f32):
    return jax.ShapeDtypeStruct(shape, dtype)


_sc_mesh = plsc.VectorSubcoreMesh(core_axis_name="c", subcore_axis_name="s")

_l1_kernel = pl.kernel(
    _l1_body,
    out_type=(_sds((NC, NP, 4, 16)), _sds((NC, NP, 16))),
    mesh=_sc_mesh,
    scratch_types=[
        pltpu.VMEM((K,), _i32), pltpu.VMEM((K,), _i32),
        pltpu.VMEM((K, 4, 16), _f32), pltpu.VMEM((K, 16), _f32),
        pltpu.VMEM((K, 16), _f32), pltpu.VMEM((K, 16), _f32),
        pltpu.VMEM_SHARED((NP, 4, 16), _f32), pltpu.VMEM_SHARED((NP, 16), _f32),
        pltpu.SemaphoreType.DMA, pltpu.SemaphoreType.DMA, pltpu.SemaphoreType.DMA,
    ],
)

_l2_kernel = pl.kernel(
    _l2_body,
    out_type=_sds((NC, NP, 16)),
    mesh=_sc_mesh,
    scratch_types=[
        pltpu.VMEM((K,), _i32), pltpu.VMEM((K,), _i32),
        pltpu.VMEM((K, 16), _f32), pltpu.VMEM((K, 16), _f32),
        pltpu.VMEM((K, 16), _f32), pltpu.VMEM((16,), _f32),
        pltpu.VMEM_SHARED((NP, 16), _f32),
        pltpu.SemaphoreType.DMA, pltpu.SemaphoreType.DMA,
    ],
)


def kernel(x, edge_index, batch, W1, a1_src, a1_dst, b1, W2, a2_src, a2_dst, b2):
    x = x.astype(_f32)
    src = edge_index[0].astype(_i32)
    dst = edge_index[1].astype(_i32)
    pad = EP - E
    src_p = jnp.concatenate([src, jnp.zeros((pad,), _i32)])
    dst_p = jnp.concatenate([dst, jnp.full((pad,), N, _i32)])

    hc = jnp.arange(HEADS * HID)
    hh = jnp.repeat(jnp.arange(HEADS), HID)
    am = jnp.zeros((HEADS * HID, 16), _f32).at[hc, hh].set(a1_src.reshape(-1))
    bm = jnp.zeros((HEADS * HID, 16), _f32).at[hc, hh].set(a1_dst.reshape(-1))
    r16 = jnp.zeros((16, HEADS * HID), _f32).at[hh, hc].set(1.0)
    ma = jnp.zeros((N_CLASSES, 16), _f32)
    ma = ma.at[:, 0].set(a2_src[0]).at[0, 1].set(1.0).at[1, 2].set(1.0)
    mb = jnp.zeros((N_CLASSES, 16), _f32).at[:, 0].set(a2_dst[0])
    c3 = jnp.zeros((1, 16), _f32).at[0, 3].set(1.0)

    h1, ca, cb = pl.pallas_call(
        _p1_body,
        out_shape=[_sds((N, HEADS * HID)), _sds((N, 16)), _sds((N, 16))],
    )(x, W1.astype(_f32), am, bm)

    z64 = jnp.zeros((NP, 4, 16), _f32)
    z16 = jnp.zeros((NP, 16), _f32)
    accp, denp = _l1_kernel(
        src_p, dst_p, h1.reshape(N, 4, 16), ca, cb, z64, z16)

    ta, tb = pl.pallas_call(
        _p3_body,
        out_shape=[_sds((N, 16)), _sds((N, 16))],
    )(accp[0, :N].reshape(N, 64), accp[1, :N].reshape(N, 64),
      denp[0, :N], denp[1, :N], r16, b1.reshape(1, -1).astype(_f32),
      W2.astype(_f32), ma, mb, c3)

    acc2 = _l2_kernel(src_p, dst_p, ta, tb, z16)

    out = pl.pallas_call(
        _p5_body,
        out_shape=_sds((N_GRAPHS, N_CLASSES)),
    )(acc2[0, :N], acc2[1, :N], batch.astype(_i32).reshape(1, N),
      b2.reshape(1, -1).astype(_f32))
    return out


# speedup vs baseline: 54.8460x; 54.8460x over previous
import functools

import jax
import jax.numpy as jnp
from jax import lax
from jax.experimental import pallas as pl
from jax.experimental.pallas import tpu as pltpu
from jax.experimental.pallas import tpu_sc as plsc

N = 10000
E = 320000
D_IN = 128
HID = 8
HEADS = 8
N_CLASSES = 2
N_GRAPHS = 64

NC = 2
NS = 16
NW = NC * NS
K = 128
NP = 10240
EP = NW * K * ((E + NW * K - 1) // (NW * K))
CHUNKS = EP // (NW * K)
RT = NP // NS

_f32 = jnp.float32
_i32 = jnp.int32


def _p1_body(x_ref, w1_ref, am_ref, bm_ref, h1_ref, ca_ref, cb_ref):
    h1 = jnp.dot(x_ref[...], w1_ref[...], preferred_element_type=_f32)
    h1_ref[...] = h1
    ca_ref[...] = jnp.dot(h1, am_ref[...], preferred_element_type=_f32)
    cb_ref[...] = jnp.dot(h1, bm_ref[...], preferred_element_type=_f32)


def _l1_body(src_hbm, dst_hbm, h1_hbm, ca_hbm, cb_hbm, z64_hbm, z16_hbm,
             acc_out, den_out,
             srcv, dstv, hrows, arows, brows, wbuf, acc_s, den_s,
             sem1, sem2, sem3):
    c = lax.axis_index("c")
    s = lax.axis_index("s")
    wid = s * NC + c
    rows0 = s * RT
    pltpu.sync_copy(z64_hbm.at[pl.ds(rows0, RT)], acc_s.at[pl.ds(rows0, RT)])
    pltpu.sync_copy(z16_hbm.at[pl.ds(rows0, RT)], den_s.at[pl.ds(rows0, RT)])
    plsc.subcore_barrier()

    lane = lax.broadcasted_iota(_i32, (16,), 0)
    qsel = lane >> 3
    base = wid * (EP // NW)

    @pl.loop(0, CHUNKS)
    def _chunk(i):
        off = base + i * K
        pltpu.sync_copy(src_hbm.at[pl.ds(off, K)], srcv)
        pltpu.sync_copy(dst_hbm.at[pl.ds(off, K)], dstv)
        ga = pltpu.async_copy(ca_hbm.at[srcv], arows, sem1)
        gb = pltpu.async_copy(cb_hbm.at[dstv], brows, sem2)
        gh = pltpu.async_copy(h1_hbm.at[srcv], hrows, sem3)
        ga.wait()
        gb.wait()
        gh.wait()

        @pl.loop(0, K)
        def _edge(e):
            t = arows[e] + brows[e]
            t = jnp.where(t >= 0, t, 0.2 * t)
            w = jnp.exp(t)
            wbuf[e] = w
            for q in range(4):
                wq = w.at[qsel + 2 * q].get(mode="promise_in_bounds")
                hrows[e, q] = hrows[e, q] * wq

        pltpu.sync_copy(wbuf, den_s.at[dstv], add=True)
        pltpu.sync_copy(hrows, acc_s.at[dstv], add=True)

    plsc.subcore_barrier()
    pltpu.sync_copy(acc_s.at[pl.ds(rows0, RT)], acc_out.at[c, pl.ds(rows0, RT)])
    pltpu.sync_copy(den_s.at[pl.ds(rows0, RT)], den_out.at[c, pl.ds(rows0, RT)])


def _p3_body(a0_ref, a1_ref, d0_ref, d1_ref, r16_ref, b1_ref, w2_ref,
             ma_ref, mb_ref, c3_ref, ta_ref, tb_ref):
    acc = a0_ref[...] + a1_ref[...]
    den = d0_ref[...] + d1_ref[...]
    deno = jnp.dot(den, r16_ref[...], preferred_element_type=_f32)
    h = acc / (deno + 1e-16) + b1_ref[...]
    h = jnp.where(h > 0, h, jnp.exp(h) - 1.0)
    h2 = jnp.dot(h, w2_ref[...], preferred_element_type=_f32)
    ta_ref[...] = jnp.dot(h2, ma_ref[...], preferred_element_type=_f32) + c3_ref[...]
    tb_ref[...] = jnp.dot(h2, mb_ref[...], preferred_element_type=_f32)


def _l2_body(src_hbm, dst_hbm, ta_hbm, tb_hbm, z16_hbm,
             acc_out,
             srcv, dstv, arows, brows, rowbuf, acc_s,
             sem1, sem2):
    c = lax.axis_index("c")
    s = lax.axis_index("s")
    wid = s * NC + c
    rows0 = s * RT
    pltpu.sync_copy(z16_hbm.at[pl.ds(rows0, RT)], acc_s.at[pl.ds(rows0, RT)])
    plsc.subcore_barrier()

    lane = lax.broadcasted_iota(_i32, (16,), 0)
    zero16 = jnp.zeros((16,), _i32)
    pat = jnp.where(lane == 0, 3, jnp.where(lane < 3, lane, 4))
    base = wid * (EP // NW)

    @pl.loop(0, CHUNKS)
    def _chunk(i):
        off = base + i * K
        pltpu.sync_copy(src_hbm.at[pl.ds(off, K)], srcv)
        pltpu.sync_copy(dst_hbm.at[pl.ds(off, K)], dstv)
        ga = pltpu.async_copy(ta_hbm.at[srcv], arows, sem1)
        gb = pltpu.async_copy(tb_hbm.at[dstv], brows, sem2)
        ga.wait()
        gb.wait()

        @pl.loop(0, K)
        def _edge(e):
            a = arows[e]
            t = a + brows[e]
            g0 = t.at[zero16].get(mode="promise_in_bounds")
            g0 = jnp.where(g0 >= 0, g0, 0.2 * g0)
            w = jnp.exp(g0)
            mult = a.at[pat].get(mode="promise_in_bounds")
            rowbuf[e] = w * mult

        pltpu.sync_copy(rowbuf, acc_s.at[dstv], add=True)

    plsc.subcore_barrier()
    pltpu.sync_copy(acc_s.at[pl.ds(rows0, RT)], acc_out.at[c, pl.ds(rows0, RT)])


def _p5_body(a0_ref, a1_ref, batch_ref, b2_ref, out_ref):
    acc = a0_ref[...] + a1_ref[...]
    den = acc[:, 0:1]
    o = acc[:, 1:3] / (den + 1e-16) + b2_ref[...]
    ids = batch_ref[...]
    g = lax.broadcasted_iota(_i32, (N_GRAPHS, 1), 0)
    mt = (g == ids).astype(_f32)
    sums = jnp.dot(mt, o, preferred_element_type=_f32)
    cnts = jnp.sum(mt, axis=1, keepdims=True)
    pooled = sums / jnp.maximum(cnts, 1.0)
    m = jnp.max(pooled, axis=1, keepdims=True)
    z = pooled - m
    out_ref[...] = z - jnp.log(jnp.sum(jnp.exp(z), axis=1, keepdims=True))


def _sds(shape, dtype=_f32):
    return jax.ShapeDtypeStruct(shape, dtype)


@functools.lru_cache(maxsize=None)
def _sc_kernels():
    mesh = plsc.VectorSubcoreMesh(core_axis_name="c", subcore_axis_name="s")
    params = pltpu.CompilerParams(use_tc_tiling_on_sc=False)
    l1 = pl.kernel(
        _l1_body,
        out_type=(_sds((NC, NP, 4, 16)), _sds((NC, NP, 16))),
        mesh=mesh,
        scratch_types=[
            pltpu.VMEM((K,), _i32), pltpu.VMEM((K,), _i32),
            pltpu.VMEM((K, 4, 16), _f32), pltpu.VMEM((K, 16), _f32),
            pltpu.VMEM((K, 16), _f32), pltpu.VMEM((K, 16), _f32),
            pltpu.VMEM_SHARED((NP, 4, 16), _f32), pltpu.VMEM_SHARED((NP, 16), _f32),
            pltpu.SemaphoreType.DMA, pltpu.SemaphoreType.DMA, pltpu.SemaphoreType.DMA,
        ],
        compiler_params=params,
    )
    l2 = pl.kernel(
        _l2_body,
        out_type=_sds((NC, NP, 16)),
        mesh=mesh,
        scratch_types=[
            pltpu.VMEM((K,), _i32), pltpu.VMEM((K,), _i32),
            pltpu.VMEM((K, 16), _f32), pltpu.VMEM((K, 16), _f32),
            pltpu.VMEM((K, 16), _f32),
            pltpu.VMEM_SHARED((NP, 16), _f32),
            pltpu.SemaphoreType.DMA, pltpu.SemaphoreType.DMA,
        ],
        compiler_params=params,
    )
    return l1, l2


def kernel(x, edge_index, batch, W1, a1_src, a1_dst, b1, W2, a2_src, a2_dst, b2):
    x = x.astype(_f32)
    src = edge_index[0].astype(_i32)
    dst = edge_index[1].astype(_i32)
    pad = EP - E
    src_p = jnp.concatenate([src, jnp.zeros((pad,), _i32)])
    dst_p = jnp.concatenate([dst, jnp.full((pad,), N, _i32)])

    hc = jnp.arange(HEADS * HID)
    hh = jnp.repeat(jnp.arange(HEADS), HID)
    am = jnp.zeros((HEADS * HID, 16), _f32).at[hc, hh].set(a1_src.reshape(-1))
    bm = jnp.zeros((HEADS * HID, 16), _f32).at[hc, hh].set(a1_dst.reshape(-1))
    r16 = jnp.zeros((16, HEADS * HID), _f32).at[hh, hc].set(1.0)
    ma = jnp.zeros((N_CLASSES, 16), _f32)
    ma = ma.at[:, 0].set(a2_src[0]).at[0, 1].set(1.0).at[1, 2].set(1.0)
    mb = jnp.zeros((N_CLASSES, 16), _f32).at[:, 0].set(a2_dst[0])
    c3 = jnp.zeros((1, 16), _f32).at[0, 3].set(1.0)

    h1, ca, cb = pl.pallas_call(
        _p1_body,
        out_shape=[_sds((N, HEADS * HID)), _sds((N, 16)), _sds((N, 16))],
    )(x, W1.astype(_f32), am, bm)

    _l1_kernel, _l2_kernel = _sc_kernels()
    z64 = jnp.zeros((NP, 4, 16), _f32)
    z16 = jnp.zeros((NP, 16), _f32)
    accp, denp = _l1_kernel(
        src_p, dst_p, h1.reshape(N, 4, 16), ca, cb, z64, z16)

    ta, tb = pl.pallas_call(
        _p3_body,
        out_shape=[_sds((N, 16)), _sds((N, 16))],
    )(accp[0, :N].reshape(N, 64), accp[1, :N].reshape(N, 64),
      denp[0, :N], denp[1, :N], r16, b1.reshape(1, -1).astype(_f32),
      W2.astype(_f32), ma, mb, c3)

    acc2 = _l2_kernel(src_p, dst_p, ta, tb, z16)

    out = pl.pallas_call(
        _p5_body,
        out_shape=_sds((N_GRAPHS, N_CLASSES)),
    )(acc2[0, :N], acc2[1, :N], batch.astype(_i32).reshape(1, N),
      b2.reshape(1, -1).astype(_f32))
    return out
